# baseline (speedup 1.0000x reference)
import functools

import jax
import jax.numpy as jnp
from jax import lax
from jax.experimental import pallas as pl
from jax.experimental.pallas import tpu as pltpu
from jax.experimental.pallas import tpu_sc as plsc

N_NODES = 10000
N_EDGES = 320000
D = 128
L = 16
NC = 2
NS = 16
NW = NC * NS
EPW = N_EDGES // NW
C = 80
NCHUNK = EPW // C
GRP = C // L


def _dist_mult_body(src_hbm, dst_hbm, node_hbm, edge_hbm, out_hbm,
                    idx_s, idx_d, scores_all,
                    head0, tail0, rel0, head1, tail1, rel1,
                    sh0, st0, sr0, sh1, st1, sr1):
    wid = lax.axis_index("c") * NS + lax.axis_index("s")
    base = wid * EPW

    pltpu.sync_copy(src_hbm.at[pl.ds(base, EPW)], idx_s)
    pltpu.sync_copy(dst_hbm.at[pl.ds(base, EPW)], idx_d)

    bufs = ((head0, tail0, rel0, sh0, st0, sr0),
            (head1, tail1, rel1, sh1, st1, sr1))

    def fire(c, b):
        head_v, tail_v, rel_v, sh, st, sr = bufs[b]
        off = c * C
        pltpu.async_copy(node_hbm.at[idx_s.at[pl.ds(off, C)]], head_v, sh)
        pltpu.async_copy(node_hbm.at[idx_d.at[pl.ds(off, C)]], tail_v, st)
        pltpu.async_copy(edge_hbm.at[pl.ds(base + off, C)], rel_v, sr)

    def wait(c, b):
        head_v, tail_v, rel_v, sh, st, sr = bufs[b]
        off = c * C
        pltpu.make_async_copy(
            node_hbm.at[idx_s.at[pl.ds(off, C)]], head_v, sh).wait()
        pltpu.make_async_copy(
            node_hbm.at[idx_d.at[pl.ds(off, C)]], tail_v, st).wait()
        pltpu.make_async_copy(
            edge_hbm.at[pl.ds(base + off, C)], rel_v, sr).wait()

    lane = lax.iota(jnp.int32, L)

    def compute(c, b):
        head_v, tail_v, rel_v = bufs[b][:3]

        lane15 = lane == (L - 1)

        def edge_body(e, _):
            acc0 = jnp.zeros((L,), jnp.float32)
            acc1 = jnp.zeros((L,), jnp.float32)
            for k in range(D // 32):
                h = plsc.bitcast(head_v[e, pl.ds(L * k, L)], jnp.bfloat16)
                t = plsc.bitcast(tail_v[e, pl.ds(L * k, L)], jnp.bfloat16)
                p0, p1 = plsc.unpack(
                    h * t, format=plsc.PackFormat.INTERLEAVED)
                acc0 = acc0 + p0 * rel_v[e, pl.ds(32 * k, L)]
                acc1 = acc1 + p1 * rel_v[e, pl.ds(32 * k + L, L)]
            s_vec = plsc.cumsum(acc0 + acc1)
            pos = jnp.full((L,), c * C + e, jnp.int32)
            plsc.store_scatter(scores_all, [pos], s_vec, mask=lane15)
            return 0

        lax.fori_loop(0, C, edge_body, 0, unroll=2)

    fire(0, 0)

    def pair_body(k, _):
        c0 = 2 * k
        fire(c0 + 1, 1)
        wait(c0, 0)
        compute(c0, 0)
        fire(c0 + 2, 0)
        wait(c0 + 1, 1)
        compute(c0 + 1, 1)
        return 0

    lax.fori_loop(0, (NCHUNK - 1) // 2, pair_body, 0)
    wait(NCHUNK - 1, 0)
    compute(NCHUNK - 1, 0)

    pltpu.sync_copy(scores_all, out_hbm.at[pl.ds(base, EPW)])


@jax.jit
def _dist_mult(src_idx, dst_idx, node_bf, edge_emb):
    mesh = plsc.VectorSubcoreMesh(
        core_axis_name="c", subcore_axis_name="s",
        num_cores=NC, num_subcores=NS)
    return pl.kernel(
        _dist_mult_body,
        out_type=jax.ShapeDtypeStruct((N_EDGES,), jnp.float32),
        mesh=mesh,
        scratch_types=[
            pltpu.VMEM((EPW,), jnp.int32),
            pltpu.VMEM((EPW,), jnp.int32),
            pltpu.VMEM((EPW,), jnp.float32),
            pltpu.VMEM((C, D // 2), jnp.int32),
            pltpu.VMEM((C, D // 2), jnp.int32),
            pltpu.VMEM((C, D), jnp.float32),
            pltpu.VMEM((C, D // 2), jnp.int32),
            pltpu.VMEM((C, D // 2), jnp.int32),
            pltpu.VMEM((C, D), jnp.float32),
            pltpu.SemaphoreType.DMA,
            pltpu.SemaphoreType.DMA,
            pltpu.SemaphoreType.DMA,
            pltpu.SemaphoreType.DMA,
            pltpu.SemaphoreType.DMA,
            pltpu.SemaphoreType.DMA,
        ],
        compiler_params=pltpu.CompilerParams(
            needs_layout_passes=False, use_tc_tiling_on_sc=False),
    )(src_idx, dst_idx, node_bf, edge_emb)


def kernel(node_emb, edge_emb, edge_index):
    src = edge_index[0].astype(jnp.int32)
    dst = edge_index[1].astype(jnp.int32)
    node_bf = (node_emb.reshape(N_NODES, D // 32, 2, L)
               .transpose(0, 1, 3, 2)
               .astype(jnp.bfloat16))
    node_i32 = lax.bitcast_convert_type(node_bf, jnp.int32)
    node_i32 = node_i32.reshape(N_NODES, D // 2)
    return _dist_mult(src, dst, node_i32, edge_emb)


# speedup vs baseline: 1.0316x; 1.0316x over previous
import functools

import jax
import jax.numpy as jnp
from jax import lax
from jax.experimental import pallas as pl
from jax.experimental.pallas import tpu as pltpu
from jax.experimental.pallas import tpu_sc as plsc

N_NODES = 10000
N_EDGES = 320000
D = 128
L = 16
NC = 2
NS = 16
NW = NC * NS
EPW = N_EDGES // NW
C = 80
NCHUNK = EPW // C
GRP = C // L


def _dist_mult_body(src_hbm, dst_hbm, node_hbm, edge_hbm, out_hbm,
                    idx_s, idx_d, scores_all,
                    head0, tail0, rel0, head1, tail1, rel1,
                    sh0, st0, sr0, sh1, st1, sr1):
    wid = lax.axis_index("c") * NS + lax.axis_index("s")
    base = wid * EPW

    pltpu.sync_copy(src_hbm.at[pl.ds(base, EPW)], idx_s)
    pltpu.sync_copy(dst_hbm.at[pl.ds(base, EPW)], idx_d)

    bufs = ((head0, tail0, rel0, sh0, st0, sr0),
            (head1, tail1, rel1, sh1, st1, sr1))

    def fire(c, b):
        head_v, tail_v, rel_v, sh, st, sr = bufs[b]
        off = c * C
        pltpu.async_copy(node_hbm.at[idx_s.at[pl.ds(off, C)]], head_v, sh)
        pltpu.async_copy(node_hbm.at[idx_d.at[pl.ds(off, C)]], tail_v, st)
        pltpu.async_copy(edge_hbm.at[pl.ds(base + off, C)], rel_v, sr)

    def wait(c, b):
        head_v, tail_v, rel_v, sh, st, sr = bufs[b]
        off = c * C
        pltpu.make_async_copy(
            node_hbm.at[idx_s.at[pl.ds(off, C)]], head_v, sh).wait()
        pltpu.make_async_copy(
            node_hbm.at[idx_d.at[pl.ds(off, C)]], tail_v, st).wait()
        pltpu.make_async_copy(
            edge_hbm.at[pl.ds(base + off, C)], rel_v, sr).wait()

    lane = lax.iota(jnp.int32, L)

    def compute(c, b):
        head_v, tail_v, rel_v = bufs[b][:3]

        lane15 = lane == (L - 1)

        def edge_body(e, carry):
            s_prev, pos_prev = carry
            plsc.store_scatter(scores_all, [pos_prev], s_prev, mask=lane15)
            acc0 = jnp.zeros((L,), jnp.float32)
            acc1 = jnp.zeros((L,), jnp.float32)
            for k in range(D // 32):
                h = plsc.bitcast(head_v[e, pl.ds(L * k, L)], jnp.bfloat16)
                t = plsc.bitcast(tail_v[e, pl.ds(L * k, L)], jnp.bfloat16)
                p0, p1 = plsc.unpack(
                    h * t, format=plsc.PackFormat.INTERLEAVED)
                acc0 = acc0 + p0 * rel_v[e, pl.ds(32 * k, L)]
                acc1 = acc1 + p1 * rel_v[e, pl.ds(32 * k + L, L)]
            s_vec = plsc.cumsum(acc0 + acc1)
            pos = jnp.full((L,), c * C + e, jnp.int32)
            return (s_vec, pos)

        s_last, pos_last = lax.fori_loop(
            0, C, edge_body,
            (jnp.zeros((L,), jnp.float32), jnp.full((L,), c * C, jnp.int32)))
        plsc.store_scatter(scores_all, [pos_last], s_last, mask=lane15)

    fire(0, 0)

    def pair_body(k, _):
        c0 = 2 * k
        fire(c0 + 1, 1)
        wait(c0, 0)
        compute(c0, 0)
        fire(c0 + 2, 0)
        wait(c0 + 1, 1)
        compute(c0 + 1, 1)
        return 0

    lax.fori_loop(0, (NCHUNK - 1) // 2, pair_body, 0)
    wait(NCHUNK - 1, 0)
    compute(NCHUNK - 1, 0)

    pltpu.sync_copy(scores_all, out_hbm.at[pl.ds(base, EPW)])


@jax.jit
def _dist_mult(src_idx, dst_idx, node_bf, edge_emb):
    mesh = plsc.VectorSubcoreMesh(
        core_axis_name="c", subcore_axis_name="s",
        num_cores=NC, num_subcores=NS)
    return pl.kernel(
        _dist_mult_body,
        out_type=jax.ShapeDtypeStruct((N_EDGES,), jnp.float32),
        mesh=mesh,
        scratch_types=[
            pltpu.VMEM((EPW,), jnp.int32),
            pltpu.VMEM((EPW,), jnp.int32),
            pltpu.VMEM((EPW,), jnp.float32),
            pltpu.VMEM((C, D // 2), jnp.int32),
            pltpu.VMEM((C, D // 2), jnp.int32),
            pltpu.VMEM((C, D), jnp.float32),
            pltpu.VMEM((C, D // 2), jnp.int32),
            pltpu.VMEM((C, D // 2), jnp.int32),
            pltpu.VMEM((C, D), jnp.float32),
            pltpu.SemaphoreType.DMA,
            pltpu.SemaphoreType.DMA,
            pltpu.SemaphoreType.DMA,
            pltpu.SemaphoreType.DMA,
            pltpu.SemaphoreType.DMA,
            pltpu.SemaphoreType.DMA,
        ],
        compiler_params=pltpu.CompilerParams(
            needs_layout_passes=False, use_tc_tiling_on_sc=False),
    )(src_idx, dst_idx, node_bf, edge_emb)


def kernel(node_emb, edge_emb, edge_index):
    src = edge_index[0].astype(jnp.int32)
    dst = edge_index[1].astype(jnp.int32)
    node_bf = (node_emb.reshape(N_NODES, D // 32, 2, L)
               .transpose(0, 1, 3, 2)
               .astype(jnp.bfloat16))
    node_i32 = lax.bitcast_convert_type(node_bf, jnp.int32)
    node_i32 = node_i32.reshape(N_NODES, D // 2)
    return _dist_mult(src, dst, node_i32, edge_emb)
